# baseline (speedup 1.0000x reference)
import functools

import jax
import jax.numpy as jnp
from jax import lax
from jax.experimental import pallas as pl
from jax.experimental.pallas import tpu as pltpu
from jax.experimental.pallas import tpu_sc as plsc

L = 1048576
NC = 2
NS = 16
LANES = 16
NW = NC * NS
PER_W = L // NW
CHUNK = 8192
HALF = CHUNK // 2
NSUB = PER_W // CHUNK


@functools.lru_cache(maxsize=1)
def _build():
    mesh = plsc.VectorSubcoreMesh(
        core_axis_name="c", subcore_axis_name="s", num_cores=NC, num_subcores=NS
    )

    @functools.partial(
        pl.kernel,
        out_type=jax.ShapeDtypeStruct((4, L), jnp.float32),
        mesh=mesh,
        scratch_types=[
            pltpu.VMEM((2, CHUNK), jnp.int32),
            pltpu.VMEM((2, 4, CHUNK), jnp.float32),
            pltpu.SemaphoreType.DMA,
            pltpu.SemaphoreType.DMA,
            pltpu.SemaphoreType.DMA,
            pltpu.SemaphoreType.DMA,
            pltpu.SemaphoreType.DMA,
            pltpu.SemaphoreType.DMA,
        ],
    )
    def _seq2tensor(
        codes_hbm, out_hbm, codes_v, out_v, si0, si1, so0, so1, so2, so3
    ):
        wid = lax.axis_index("s") * NC + lax.axis_index("c")
        base = wid * PER_W
        in_sems = (si0, si1)
        out_sems = (so0, so1, so2, so3)

        in_copies = [None] * NSUB
        out_copies = [None] * (2 * NSUB)
        in_copies[0] = pltpu.async_copy(
            codes_hbm.at[pl.ds(base, CHUNK)], codes_v.at[0], in_sems[0]
        )
        for sub in range(NSUB):
            b = sub % 2
            if sub + 1 < NSUB:
                nb = (sub + 1) % 2
                in_copies[sub + 1] = pltpu.async_copy(
                    codes_hbm.at[pl.ds(base + (sub + 1) * CHUNK, CHUNK)],
                    codes_v.at[nb],
                    in_sems[nb],
                )
            in_copies[sub].wait()
            cv = codes_v.at[b]
            ov = out_v.at[b]
            for h in range(2):
                hidx = 2 * sub + h
                if hidx >= 4:
                    out_copies[hidx - 4].wait()

                @plsc.parallel_loop(h * HALF, (h + 1) * HALF, step=LANES, unroll=1)
                def body(i, cv=cv, ov=ov):
                    c = cv[pl.ds(i, LANES)]
                    n_fill = jnp.where(c == 4, 0.25, 0.0)
                    for r in range(4):
                        ov[r, pl.ds(i, LANES)] = jnp.where(c == r, 1.0, n_fill)

                out_copies[hidx] = pltpu.async_copy(
                    out_v.at[b, :, pl.ds(h * HALF, HALF)],
                    out_hbm.at[:, pl.ds(base + sub * CHUNK + h * HALF, HALF)],
                    out_sems[hidx % 4],
                )
        for hidx in range(2 * NSUB - 4, 2 * NSUB):
            out_copies[hidx].wait()

    return _seq2tensor


def kernel(codes):
    return _build()(codes)


# speedup vs baseline: 1.0923x; 1.0923x over previous
import functools

import jax
import jax.numpy as jnp
from jax import lax
from jax.experimental import pallas as pl
from jax.experimental.pallas import tpu as pltpu
from jax.experimental.pallas import tpu_sc as plsc

L = 1048576
NC = 2
NS = 16
LANES = 16
NW = NC * NS
PER_W = L // NW
CHUNK = 8192
HALF = CHUNK // 2
NSUB = PER_W // CHUNK


@functools.lru_cache(maxsize=1)
def _build():
    mesh = plsc.VectorSubcoreMesh(
        core_axis_name="c", subcore_axis_name="s", num_cores=NC, num_subcores=NS
    )

    @functools.partial(
        pl.kernel,
        out_type=jax.ShapeDtypeStruct((4, L), jnp.float32),
        mesh=mesh,
        scratch_types=[
            pltpu.VMEM((2, CHUNK), jnp.int32),
            pltpu.VMEM((2, 4, CHUNK), jnp.float32),
            pltpu.SemaphoreType.DMA,
            pltpu.SemaphoreType.DMA,
            pltpu.SemaphoreType.DMA,
            pltpu.SemaphoreType.DMA,
            pltpu.SemaphoreType.DMA,
            pltpu.SemaphoreType.DMA,
        ],
    )
    def _seq2tensor(
        codes_hbm, out_hbm, codes_v, out_v, si0, si1, so0, so1, so2, so3
    ):
        wid = lax.axis_index("s") * NC + lax.axis_index("c")
        base = wid * PER_W
        in_sems = (si0, si1)
        out_sems = (so0, so1, so2, so3)

        in_copies = [None] * NSUB
        out_copies = [None] * (2 * NSUB)
        in_copies[0] = pltpu.async_copy(
            codes_hbm.at[pl.ds(base, CHUNK)], codes_v.at[0], in_sems[0]
        )
        for sub in range(NSUB):
            b = sub % 2
            if sub + 1 < NSUB:
                nb = (sub + 1) % 2
                in_copies[sub + 1] = pltpu.async_copy(
                    codes_hbm.at[pl.ds(base + (sub + 1) * CHUNK, CHUNK)],
                    codes_v.at[nb],
                    in_sems[nb],
                )
            in_copies[sub].wait()
            cv = codes_v.at[b]
            ov = out_v.at[b]
            for h in range(2):
                hidx = 2 * sub + h
                if hidx >= 4:
                    out_copies[hidx - 4].wait()

                @plsc.parallel_loop(h * HALF, (h + 1) * HALF, step=LANES, unroll=2)
                def body(i, cv=cv, ov=ov):
                    c = cv[pl.ds(i, LANES)]
                    n_fill = jnp.where(c == 4, 0.25, 0.0)
                    for r in range(4):
                        ov[r, pl.ds(i, LANES)] = jnp.where(c == r, 1.0, n_fill)

                out_copies[hidx] = pltpu.async_copy(
                    out_v.at[b, :, pl.ds(h * HALF, HALF)],
                    out_hbm.at[:, pl.ds(base + sub * CHUNK + h * HALF, HALF)],
                    out_sems[hidx % 4],
                )
        for hidx in range(2 * NSUB - 4, 2 * NSUB):
            out_copies[hidx].wait()

    return _seq2tensor


def kernel(codes):
    return _build()(codes)
